# baseline (speedup 1.0000x reference)
import functools

import jax
import jax.numpy as jnp
from jax import lax
from jax.experimental import pallas as pl
from jax.experimental.pallas import tpu as pltpu
from jax.experimental.pallas import tpu_sc as plsc

NPTS = 8192
KNBR = 20
SLOPE = 0.2
EPS = 1e-5
GW = 128


_BR = 128


_NG = 512
_NT = NPTS // _NG
_INF = jnp.inf


def _insert3(d, c, r1, i1, r2, i2, r3, i3):
    c1 = d < r1
    c2 = d < r2
    c3 = d < r3
    r1n = jnp.where(c1, d, r1)
    i1n = jnp.where(c1, c, i1)
    r2n = jnp.where(c1, r1, jnp.where(c2, d, r2))
    i2n = jnp.where(c1, i1, jnp.where(c2, c, i2))
    r3n = jnp.where(c2, r2, jnp.where(c3, d, r3))
    i3n = jnp.where(c2, i2, jnp.where(c3, c, i3))
    return r1n, i1n, r2n, i2n, r3n, i3n


def _knn_body(x_ref, xt_ref, idx_ref, d_ref):
    xt = xt_ref[...]
    xb = x_ref[...]
    dot = lax.dot_general(
        xb, xt, (((1,), (0,)), ((), ())),
        preferred_element_type=jnp.float32)
    sqr = jnp.sum(xb * xb, axis=1, keepdims=True)
    sqc = jnp.sum(xt * xt, axis=0, keepdims=True)
    d = (sqr - 2.0 * dot) + sqc
    d_ref[...] = d

    giota = lax.broadcasted_iota(jnp.int32, (_BR, _NG), 1)
    oiota = lax.broadcasted_iota(jnp.int32, (_BR, KNBR), 1)
    shape = (_BR, _NG)

    def build(excl_v, excl_i):
        def bt(t, st):
            r1, i1, r2, i2, r3, i3 = st
            base = pl.multiple_of(t * _NG, _NG)
            dt = d_ref[:, pl.ds(base, _NG)]
            ct = giota + t * _NG
            live = (dt > excl_v) | ((dt == excl_v) & (ct > excl_i))
            dt = jnp.where(live, dt, _INF)
            return _insert3(dt, ct, r1, i1, r2, i2, r3, i3)

        init = (jnp.full(shape, _INF, jnp.float32), jnp.zeros(shape, jnp.int32),
                jnp.full(shape, _INF, jnp.float32), jnp.zeros(shape, jnp.int32),
                jnp.full(shape, _INF, jnp.float32), jnp.zeros(shape, jnp.int32))
        return lax.fori_loop(0, _NT, bt, init)

    st0 = build(jnp.full((_BR, 1), -_INF, jnp.float32),
                jnp.full((_BR, 1), -1, jnp.int32))

    def pop_body(j, carry):
        r1, i1, r2, i2, r3, i3, out = carry
        m = jnp.min(r1, axis=1, keepdims=True)
        istar = jnp.min(jnp.where(r1 == m, i1, NPTS), axis=1, keepdims=True)
        out = jnp.where(oiota == j, istar, out)
        sel = (r1 == m) & (i1 == istar)
        trigger = jnp.max(jnp.where(sel & (r2 == _INF), 1, 0))
        r1 = jnp.where(sel, r2, r1)
        i1 = jnp.where(sel, i2, i1)
        r2 = jnp.where(sel, r3, r2)
        i2 = jnp.where(sel, i3, i2)
        r3 = jnp.where(sel, _INF, r3)
        st = lax.cond(trigger > 0,
                      lambda: build(m, istar),
                      lambda: (r1, i1, r2, i2, r3, i3))
        return (*st, out)

    carry0 = (*st0, jnp.zeros((_BR, KNBR), jnp.int32))
    res = lax.fori_loop(0, KNBR, pop_body, carry0)
    idx_ref[...] = res[6]


def _knn(xp, xt):
    n, c = xp.shape
    return pl.pallas_call(
        _knn_body,
        grid=(n // _BR,),
        in_specs=[pl.BlockSpec((_BR, c), lambda i: (i, 0)),
                  pl.BlockSpec((c, n), lambda i: (0, 0))],
        out_specs=pl.BlockSpec((_BR, KNBR), lambda i: (i, 0)),
        out_shape=jax.ShapeDtypeStruct((n, KNBR), jnp.int32),
        scratch_shapes=[pltpu.VMEM((_BR, n), jnp.float32)],
    )(xp, xt)


def _sc_gather(table, idx_flat):
    n, width = table.shape
    ne = n * KNBR
    info = plsc.get_sparse_core_info()
    ncores = info.num_cores
    nw = ncores * info.num_subcores
    per_w = ne // nw
    eb = 512
    nch = per_w // eb
    mesh = plsc.VectorSubcoreMesh(core_axis_name="c", subcore_axis_name="s")

    @functools.partial(
        pl.kernel, mesh=mesh,
        out_type=jax.ShapeDtypeStruct((ne, width), jnp.float32),
        scratch_types=[
            pltpu.VMEM((eb,), jnp.int32),
            pltpu.VMEM((eb, width), jnp.float32),
            pltpu.SemaphoreType.DMA,
        ])
    def k(tab_hbm, idx_hbm, out_hbm, idx_v, rows_v, sem):
        wid = lax.axis_index("s") * ncores + lax.axis_index("c")

        def chunk_body(ci, carry):
            base = wid * per_w + ci * eb
            pltpu.sync_copy(idx_hbm.at[pl.ds(base, eb)], idx_v)
            pltpu.async_copy(tab_hbm.at[idx_v], rows_v, sem).wait()
            pltpu.sync_copy(rows_v, out_hbm.at[pl.ds(base, eb)])
            return carry

        lax.fori_loop(0, nch, chunk_body, 0)

    return k(table, idx_flat)


def _edge_body(cin, x_ref, nbr_ref, w_ref, b_ref, mx_ref, mn_ref, st_ref):
    xs = x_ref[...][:, :cin]
    nb = nbr_ref[...][:, :cin]
    crep = jnp.broadcast_to(
        jnp.reshape(xs, (_BR, 1, cin)), (_BR, KNBR, cin))
    crep = jnp.reshape(crep, (_BR * KNBR, cin))
    e_in = jnp.concatenate([crep, nb - crep], axis=1)
    e = lax.dot_general(
        e_in, w_ref[...], (((1,), (0,)), ((), ())),
        preferred_element_type=jnp.float32) + b_ref[...]

    @pl.when(pl.program_id(0) == 0)
    def _():
        st_ref[...] = jnp.zeros_like(st_ref)

    st_ref[...] += jnp.concatenate(
        [jnp.sum(e, axis=0, keepdims=True),
         jnp.sum(e * e, axis=0, keepdims=True)], axis=0)
    e3 = jnp.reshape(e, (_BR, KNBR, e.shape[1]))
    mx_ref[...] = jnp.max(e3, axis=1)
    mn_ref[...] = jnp.min(e3, axis=1)


def _edge_fused(xg, nbr, w, b, cin):
    n = xg.shape[0]
    cout = w.shape[1]
    body = functools.partial(_edge_body, cin)
    return pl.pallas_call(
        body,
        grid=(n // _BR,),
        in_specs=[pl.BlockSpec((_BR, GW), lambda i: (i, 0)),
                  pl.BlockSpec((_BR * KNBR, GW), lambda i: (i, 0)),
                  pl.BlockSpec((2 * cin, cout), lambda i: (0, 0)),
                  pl.BlockSpec((1, cout), lambda i: (0, 0))],
        out_specs=[pl.BlockSpec((_BR, cout), lambda i: (i, 0)),
                   pl.BlockSpec((_BR, cout), lambda i: (i, 0)),
                   pl.BlockSpec((2, cout), lambda i: (0, 0))],
        out_shape=[jax.ShapeDtypeStruct((n, cout), jnp.float32),
                   jax.ShapeDtypeStruct((n, cout), jnp.float32),
                   jax.ShapeDtypeStruct((2, cout), jnp.float32)],
    )(xg, nbr, w, b.reshape(1, cout))


_BO = 1024


def _out_body(mx_ref, mn_ref, st_ref, g_ref, be_ref, o_ref):
    st = st_ref[...]
    nk = float(NPTS * KNBR)
    mu = st[0:1] / nk
    var = st[1:2] / nk - mu * mu
    g = g_ref[...]
    m = jnp.where(g >= 0, mx_ref[...], mn_ref[...])
    z = (m - mu) / jnp.sqrt(var + EPS) * g + be_ref[...]
    o_ref[...] = jnp.where(z >= 0, z, SLOPE * z)


def _edge_out(mx, mn, st, g, be):
    n, cout = mx.shape
    row = pl.BlockSpec((_BO, cout), lambda i: (i, 0))
    vec = pl.BlockSpec((1, cout), lambda i: (0, 0))
    return pl.pallas_call(
        _out_body,
        grid=(n // _BO,),
        in_specs=[row, row, pl.BlockSpec((2, cout), lambda i: (0, 0)),
                  vec, vec],
        out_specs=row,
        out_shape=jax.ShapeDtypeStruct((n, cout), jnp.float32),
    )(mx, mn, st, g.reshape(1, cout), be.reshape(1, cout))


def _final_linear_body(x1_ref, x2_ref, x3_ref, x4_ref, w_ref, b_ref, o_ref):
    feat = jnp.concatenate(
        [x1_ref[...], x2_ref[...], x3_ref[...], x4_ref[...]], axis=1)
    o_ref[...] = lax.dot_general(
        feat, w_ref[...], (((1,), (0,)), ((), ())),
        preferred_element_type=jnp.float32) + b_ref[...]


def _final_linear(x1, x2, x3, x4, w5, b5, br=512):
    n = x1.shape[0]
    m = w5.shape[1]
    cat = w5.shape[0]

    def rowspec(c):
        return pl.BlockSpec((br, c), lambda i: (i, 0))

    return pl.pallas_call(
        _final_linear_body,
        grid=(n // br,),
        in_specs=[rowspec(x1.shape[1]), rowspec(x2.shape[1]),
                  rowspec(x3.shape[1]), rowspec(x4.shape[1]),
                  pl.BlockSpec((cat, m), lambda i: (0, 0)),
                  pl.BlockSpec((1, m), lambda i: (0, 0))],
        out_specs=pl.BlockSpec((br, m), lambda i: (i, 0)),
        out_shape=jax.ShapeDtypeStruct((n, m), jnp.float32),
    )(x1, x2, x3, x4, w5, b5.reshape(1, m))


def _edge_layer(x, w, bias, g, be):
    n, cin = x.shape
    if cin < GW:
        xg = jnp.concatenate(
            [x, jnp.zeros((n, GW - cin), jnp.float32)], axis=1)
    else:
        xg = x
    idx = _knn(xg, xg.T)
    nbr = _sc_gather(xg, idx.reshape(-1))
    mx, mn, st = _edge_fused(xg, nbr, w, bias, cin)
    return _edge_out(mx, mn, st, g, be)


def kernel(x, W1, b1, g1, be1, W2, b2, g2, be2, W3, b3, g3, be3,
           W4, b4, g4, be4, W5, b5):
    x1 = _edge_layer(x, W1, b1, g1, be1)
    x2 = _edge_layer(x1, W2, b2, g2, be2)
    x3 = _edge_layer(x2, W3, b3, g3, be3)
    x4 = _edge_layer(x3, W4, b4, g4, be4)
    return _final_linear(x1, x2, x3, x4, W5, b5)


# speedup vs baseline: 1.4445x; 1.4445x over previous
import functools

import jax
import jax.numpy as jnp
from jax import lax
from jax.experimental import pallas as pl
from jax.experimental.pallas import tpu as pltpu
from jax.experimental.pallas import tpu_sc as plsc

NPTS = 8192
KNBR = 20
SLOPE = 0.2
EPS = 1e-5
GW = 128


_BR = 128


_NG = 512
_NT = NPTS // _NG
_INF = jnp.inf


def _insert3(d, c, r1, i1, r2, i2, r3, i3):
    c1 = d < r1
    c2 = d < r2
    c3 = d < r3
    r1n = jnp.where(c1, d, r1)
    i1n = jnp.where(c1, c, i1)
    r2n = jnp.where(c1, r1, jnp.where(c2, d, r2))
    i2n = jnp.where(c1, i1, jnp.where(c2, c, i2))
    r3n = jnp.where(c2, r2, jnp.where(c3, d, r3))
    i3n = jnp.where(c2, i2, jnp.where(c3, c, i3))
    return r1n, i1n, r2n, i2n, r3n, i3n


def _knn_body(x_ref, xt_ref, idx_ref, d_ref):
    xt = xt_ref[...]
    xb = x_ref[...]
    dot = lax.dot_general(
        xb, xt, (((1,), (0,)), ((), ())),
        preferred_element_type=jnp.float32)
    sqr = jnp.sum(xb * xb, axis=1, keepdims=True)
    sqc = jnp.sum(xt * xt, axis=0, keepdims=True)
    d = (sqr - 2.0 * dot) + sqc
    d_ref[...] = d

    giota = lax.broadcasted_iota(jnp.int32, (_BR, _NG), 1)
    shape = (_BR, _NG)

    def rebuild(excl_v, excl_i):
        def bt(t, st):
            r1, i1, r2, i2, r3, i3 = st
            base = pl.multiple_of(t * _NG, _NG)
            dt = d_ref[:, pl.ds(base, _NG)]
            ct = giota + t * _NG
            live = (dt > excl_v) | ((dt == excl_v) & (ct > excl_i))
            dt = jnp.where(live, dt, _INF)
            return _insert3(dt, ct, r1, i1, r2, i2, r3, i3)

        init = (jnp.full(shape, _INF, jnp.float32), jnp.zeros(shape, jnp.int32),
                jnp.full(shape, _INF, jnp.float32), jnp.zeros(shape, jnp.int32),
                jnp.full(shape, _INF, jnp.float32), jnp.zeros(shape, jnp.int32))
        return lax.fori_loop(0, _NT, bt, init)

    r1 = jnp.full(shape, _INF, jnp.float32)
    r2 = jnp.full(shape, _INF, jnp.float32)
    r3 = jnp.full(shape, _INF, jnp.float32)
    i1 = jnp.zeros(shape, jnp.int32)
    i2 = jnp.zeros(shape, jnp.int32)
    i3 = jnp.zeros(shape, jnp.int32)
    for t in range(_NT):
        dt = d[:, t * _NG:(t + 1) * _NG]
        ct = giota + t * _NG
        r1, i1, r2, i2, r3, i3 = _insert3(dt, ct, r1, i1, r2, i2, r3, i3)

    outs = []
    for _ in range(KNBR):
        m = jnp.min(r1, axis=1, keepdims=True)
        istar = jnp.min(jnp.where(r1 == m, i1, NPTS), axis=1, keepdims=True)
        outs.append(istar)
        sel = (r1 == m) & (i1 == istar)
        trigger = jnp.max(jnp.where(sel & (r2 == _INF), 1, 0))
        r1 = jnp.where(sel, r2, r1)
        i1 = jnp.where(sel, i2, i1)
        r2 = jnp.where(sel, r3, r2)
        i2 = jnp.where(sel, i3, i2)
        r3 = jnp.where(sel, _INF, r3)
        st = (r1, i1, r2, i2, r3, i3)
        r1, i1, r2, i2, r3, i3 = lax.cond(
            trigger > 0, functools.partial(rebuild, m, istar),
            lambda st=st: st)

    idx_ref[...] = jnp.concatenate(outs, axis=1)


def _knn(xp, xt):
    n, c = xp.shape
    return pl.pallas_call(
        _knn_body,
        grid=(n // _BR,),
        in_specs=[pl.BlockSpec((_BR, c), lambda i: (i, 0)),
                  pl.BlockSpec((c, n), lambda i: (0, 0))],
        out_specs=pl.BlockSpec((_BR, KNBR), lambda i: (i, 0)),
        out_shape=jax.ShapeDtypeStruct((n, KNBR), jnp.int32),
        scratch_shapes=[pltpu.VMEM((_BR, n), jnp.float32)],
    )(xp, xt)


def _sc_gather(table, idx_flat):
    n, width = table.shape
    ne = n * KNBR
    info = plsc.get_sparse_core_info()
    ncores = info.num_cores
    nw = ncores * info.num_subcores
    per_w = ne // nw
    eb = 512
    nch = per_w // eb
    mesh = plsc.VectorSubcoreMesh(core_axis_name="c", subcore_axis_name="s")

    @functools.partial(
        pl.kernel, mesh=mesh,
        out_type=jax.ShapeDtypeStruct((ne, width), jnp.float32),
        scratch_types=[
            pltpu.VMEM((eb,), jnp.int32),
            pltpu.VMEM((eb, width), jnp.float32),
            pltpu.SemaphoreType.DMA,
        ])
    def k(tab_hbm, idx_hbm, out_hbm, idx_v, rows_v, sem):
        wid = lax.axis_index("s") * ncores + lax.axis_index("c")

        def chunk_body(ci, carry):
            base = wid * per_w + ci * eb
            pltpu.sync_copy(idx_hbm.at[pl.ds(base, eb)], idx_v)
            pltpu.async_copy(tab_hbm.at[idx_v], rows_v, sem).wait()
            pltpu.sync_copy(rows_v, out_hbm.at[pl.ds(base, eb)])
            return carry

        lax.fori_loop(0, nch, chunk_body, 0)

    return k(table, idx_flat)


def _edge_body(cin, x_ref, nbr_ref, w_ref, b_ref, mx_ref, mn_ref, st_ref):
    xs = x_ref[...][:, :cin]
    nb = nbr_ref[...][:, :cin]
    crep = jnp.broadcast_to(
        jnp.reshape(xs, (_BR, 1, cin)), (_BR, KNBR, cin))
    crep = jnp.reshape(crep, (_BR * KNBR, cin))
    e_in = jnp.concatenate([crep, nb - crep], axis=1)
    e = lax.dot_general(
        e_in, w_ref[...], (((1,), (0,)), ((), ())),
        preferred_element_type=jnp.float32) + b_ref[...]

    @pl.when(pl.program_id(0) == 0)
    def _():
        st_ref[...] = jnp.zeros_like(st_ref)

    st_ref[...] += jnp.concatenate(
        [jnp.sum(e, axis=0, keepdims=True),
         jnp.sum(e * e, axis=0, keepdims=True)], axis=0)
    e3 = jnp.reshape(e, (_BR, KNBR, e.shape[1]))
    mx_ref[...] = jnp.max(e3, axis=1)
    mn_ref[...] = jnp.min(e3, axis=1)


def _edge_fused(xg, nbr, w, b, cin):
    n = xg.shape[0]
    cout = w.shape[1]
    body = functools.partial(_edge_body, cin)
    return pl.pallas_call(
        body,
        grid=(n // _BR,),
        in_specs=[pl.BlockSpec((_BR, GW), lambda i: (i, 0)),
                  pl.BlockSpec((_BR * KNBR, GW), lambda i: (i, 0)),
                  pl.BlockSpec((2 * cin, cout), lambda i: (0, 0)),
                  pl.BlockSpec((1, cout), lambda i: (0, 0))],
        out_specs=[pl.BlockSpec((_BR, cout), lambda i: (i, 0)),
                   pl.BlockSpec((_BR, cout), lambda i: (i, 0)),
                   pl.BlockSpec((2, cout), lambda i: (0, 0))],
        out_shape=[jax.ShapeDtypeStruct((n, cout), jnp.float32),
                   jax.ShapeDtypeStruct((n, cout), jnp.float32),
                   jax.ShapeDtypeStruct((2, cout), jnp.float32)],
    )(xg, nbr, w, b.reshape(1, cout))


_BO = 1024


def _out_body(mx_ref, mn_ref, st_ref, g_ref, be_ref, o_ref):
    st = st_ref[...]
    nk = float(NPTS * KNBR)
    mu = st[0:1] / nk
    var = st[1:2] / nk - mu * mu
    g = g_ref[...]
    m = jnp.where(g >= 0, mx_ref[...], mn_ref[...])
    z = (m - mu) / jnp.sqrt(var + EPS) * g + be_ref[...]
    o_ref[...] = jnp.where(z >= 0, z, SLOPE * z)


def _edge_out(mx, mn, st, g, be):
    n, cout = mx.shape
    row = pl.BlockSpec((_BO, cout), lambda i: (i, 0))
    vec = pl.BlockSpec((1, cout), lambda i: (0, 0))
    return pl.pallas_call(
        _out_body,
        grid=(n // _BO,),
        in_specs=[row, row, pl.BlockSpec((2, cout), lambda i: (0, 0)),
                  vec, vec],
        out_specs=row,
        out_shape=jax.ShapeDtypeStruct((n, cout), jnp.float32),
    )(mx, mn, st, g.reshape(1, cout), be.reshape(1, cout))


def _final_linear_body(x1_ref, x2_ref, x3_ref, x4_ref, w_ref, b_ref, o_ref):
    feat = jnp.concatenate(
        [x1_ref[...], x2_ref[...], x3_ref[...], x4_ref[...]], axis=1)
    o_ref[...] = lax.dot_general(
        feat, w_ref[...], (((1,), (0,)), ((), ())),
        preferred_element_type=jnp.float32) + b_ref[...]


def _final_linear(x1, x2, x3, x4, w5, b5, br=512):
    n = x1.shape[0]
    m = w5.shape[1]
    cat = w5.shape[0]

    def rowspec(c):
        return pl.BlockSpec((br, c), lambda i: (i, 0))

    return pl.pallas_call(
        _final_linear_body,
        grid=(n // br,),
        in_specs=[rowspec(x1.shape[1]), rowspec(x2.shape[1]),
                  rowspec(x3.shape[1]), rowspec(x4.shape[1]),
                  pl.BlockSpec((cat, m), lambda i: (0, 0)),
                  pl.BlockSpec((1, m), lambda i: (0, 0))],
        out_specs=pl.BlockSpec((br, m), lambda i: (i, 0)),
        out_shape=jax.ShapeDtypeStruct((n, m), jnp.float32),
    )(x1, x2, x3, x4, w5, b5.reshape(1, m))


def _edge_layer(x, w, bias, g, be):
    n, cin = x.shape
    if cin < GW:
        xg = jnp.concatenate(
            [x, jnp.zeros((n, GW - cin), jnp.float32)], axis=1)
    else:
        xg = x
    idx = _knn(xg, xg.T)
    nbr = _sc_gather(xg, idx.reshape(-1))
    mx, mn, st = _edge_fused(xg, nbr, w, bias, cin)
    return _edge_out(mx, mn, st, g, be)


def kernel(x, W1, b1, g1, be1, W2, b2, g2, be2, W3, b3, g3, be3,
           W4, b4, g4, be4, W5, b5):
    x1 = _edge_layer(x, W1, b1, g1, be1)
    x2 = _edge_layer(x1, W2, b2, g2, be2)
    x3 = _edge_layer(x2, W3, b3, g3, be3)
    x4 = _edge_layer(x3, W4, b4, g4, be4)
    return _final_linear(x1, x2, x3, x4, W5, b5)


# speedup vs baseline: 2.3043x; 1.5952x over previous
import functools

import jax
import jax.numpy as jnp
from jax import lax
from jax.experimental import pallas as pl
from jax.experimental.pallas import tpu as pltpu
from jax.experimental.pallas import tpu_sc as plsc

NPTS = 8192
KNBR = 20
SLOPE = 0.2
EPS = 1e-5
GW = 128


_BR = 128


_NG = 512
_NT = NPTS // _NG
_RS = 16
_INF = jnp.inf


def _insert3(d, c, r1, i1, r2, i2, r3, i3):
    c1 = d < r1
    c2 = d < r2
    c3 = d < r3
    r1n = jnp.where(c1, d, r1)
    i1n = jnp.where(c1, c, i1)
    r2n = jnp.where(c1, r1, jnp.where(c2, d, r2))
    i2n = jnp.where(c1, i1, jnp.where(c2, c, i2))
    r3n = jnp.where(c2, r2, jnp.where(c3, d, r3))
    i3n = jnp.where(c2, i2, jnp.where(c3, c, i3))
    return r1n, i1n, r2n, i2n, r3n, i3n


def _knn_body(x_ref, xt_ref, idx_ref, d_ref):
    xt = xt_ref[...]
    xb = x_ref[...]
    dot = lax.dot_general(
        xb, xt, (((1,), (0,)), ((), ())),
        preferred_element_type=jnp.float32)
    sqr = jnp.sum(xb * xb, axis=1, keepdims=True)
    sqc = jnp.sum(xt * xt, axis=0, keepdims=True)
    d = (sqr - 2.0 * dot) + sqc
    d_ref[...] = d

    giota = lax.broadcasted_iota(jnp.int32, (_RS, _NG), 1)
    oiota = lax.broadcasted_iota(jnp.int32, (_RS, KNBR), 1)
    shape = (_RS, _NG)

    slices = []
    for rs in range(0, _BR, _RS):
        r1 = jnp.full(shape, _INF, jnp.float32)
        r2 = jnp.full(shape, _INF, jnp.float32)
        r3 = jnp.full(shape, _INF, jnp.float32)
        i1 = jnp.zeros(shape, jnp.int32)
        i2 = jnp.zeros(shape, jnp.int32)
        i3 = jnp.zeros(shape, jnp.int32)
        for t in range(_NT):
            dt = d_ref[rs:rs + _RS, t * _NG:(t + 1) * _NG]
            ct = giota + t * _NG
            r1, i1, r2, i2, r3, i3 = _insert3(dt, ct, r1, i1, r2, i2, r3, i3)

        outs = []
        trig = jnp.int32(0)
        for _ in range(KNBR):
            m = jnp.min(r1, axis=1, keepdims=True)
            istar = jnp.min(jnp.where(r1 == m, i1, NPTS), axis=1,
                            keepdims=True)
            outs.append(istar)
            sel = (r1 == m) & (i1 == istar)
            trig = jnp.maximum(
                trig, jnp.max(jnp.where(sel & (r2 == _INF), 1, 0)))
            r1 = jnp.where(sel, r2, r1)
            i1 = jnp.where(sel, i2, i1)
            r2 = jnp.where(sel, r3, r2)
            i2 = jnp.where(sel, i3, i2)
            r3 = jnp.where(sel, _INF, r3)
        slices.append((rs, jnp.concatenate(outs, axis=1), trig))

    for rs, idx_slice, trig in slices:
        def _fallback(rs=rs):
            cols = lax.broadcasted_iota(jnp.int32, (_RS, NPTS), 1)

            def fb(j, carry):
                dd, out = carry
                mm = jnp.min(dd, axis=1, keepdims=True)
                ii = jnp.min(jnp.where(dd == mm, cols, NPTS), axis=1,
                             keepdims=True)
                out = jnp.where(oiota == j, ii, out)
                dd = jnp.where(cols == ii, _INF, dd)
                return dd, out

            dd0 = d_ref[rs:rs + _RS, :]
            _, out = lax.fori_loop(
                0, KNBR, fb, (dd0, jnp.zeros((_RS, KNBR), jnp.int32)))
            return out

        idx_slice = lax.cond(trig > 0, _fallback,
                             lambda idx_slice=idx_slice: idx_slice)
        idx_ref[rs:rs + _RS, :] = idx_slice


def _knn(xp, xt):
    n, c = xp.shape
    return pl.pallas_call(
        _knn_body,
        grid=(n // _BR,),
        in_specs=[pl.BlockSpec((_BR, c), lambda i: (i, 0)),
                  pl.BlockSpec((c, n), lambda i: (0, 0))],
        out_specs=pl.BlockSpec((_BR, KNBR), lambda i: (i, 0)),
        out_shape=jax.ShapeDtypeStruct((n, KNBR), jnp.int32),
        scratch_shapes=[pltpu.VMEM((_BR, n), jnp.float32)],
    )(xp, xt)


def _sc_gather(table, idx_flat):
    n, width = table.shape
    ne = n * KNBR
    info = plsc.get_sparse_core_info()
    ncores = info.num_cores
    nw = ncores * info.num_subcores
    per_w = ne // nw
    eb = 512
    nch = per_w // eb
    mesh = plsc.VectorSubcoreMesh(core_axis_name="c", subcore_axis_name="s")

    @functools.partial(
        pl.kernel, mesh=mesh,
        out_type=jax.ShapeDtypeStruct((ne, width), jnp.float32),
        scratch_types=[
            pltpu.VMEM((eb,), jnp.int32),
            pltpu.VMEM((eb, width), jnp.float32),
            pltpu.SemaphoreType.DMA,
        ])
    def k(tab_hbm, idx_hbm, out_hbm, idx_v, rows_v, sem):
        wid = lax.axis_index("s") * ncores + lax.axis_index("c")

        def chunk_body(ci, carry):
            base = wid * per_w + ci * eb
            pltpu.sync_copy(idx_hbm.at[pl.ds(base, eb)], idx_v)
            pltpu.async_copy(tab_hbm.at[idx_v], rows_v, sem).wait()
            pltpu.sync_copy(rows_v, out_hbm.at[pl.ds(base, eb)])
            return carry

        lax.fori_loop(0, nch, chunk_body, 0)

    return k(table, idx_flat)


def _edge_body(cin, x_ref, nbr_ref, w_ref, b_ref, mx_ref, mn_ref, st_ref):
    xs = x_ref[...][:, :cin]
    nb = nbr_ref[...][:, :cin]
    crep = jnp.broadcast_to(
        jnp.reshape(xs, (_BR, 1, cin)), (_BR, KNBR, cin))
    crep = jnp.reshape(crep, (_BR * KNBR, cin))
    e_in = jnp.concatenate([crep, nb - crep], axis=1)
    e = lax.dot_general(
        e_in, w_ref[...], (((1,), (0,)), ((), ())),
        preferred_element_type=jnp.float32) + b_ref[...]

    @pl.when(pl.program_id(0) == 0)
    def _():
        st_ref[...] = jnp.zeros_like(st_ref)

    st_ref[...] += jnp.concatenate(
        [jnp.sum(e, axis=0, keepdims=True),
         jnp.sum(e * e, axis=0, keepdims=True)], axis=0)
    e3 = jnp.reshape(e, (_BR, KNBR, e.shape[1]))
    mx_ref[...] = jnp.max(e3, axis=1)
    mn_ref[...] = jnp.min(e3, axis=1)


def _edge_fused(xg, nbr, w, b, cin):
    n = xg.shape[0]
    cout = w.shape[1]
    body = functools.partial(_edge_body, cin)
    return pl.pallas_call(
        body,
        grid=(n // _BR,),
        in_specs=[pl.BlockSpec((_BR, GW), lambda i: (i, 0)),
                  pl.BlockSpec((_BR * KNBR, GW), lambda i: (i, 0)),
                  pl.BlockSpec((2 * cin, cout), lambda i: (0, 0)),
                  pl.BlockSpec((1, cout), lambda i: (0, 0))],
        out_specs=[pl.BlockSpec((_BR, cout), lambda i: (i, 0)),
                   pl.BlockSpec((_BR, cout), lambda i: (i, 0)),
                   pl.BlockSpec((2, cout), lambda i: (0, 0))],
        out_shape=[jax.ShapeDtypeStruct((n, cout), jnp.float32),
                   jax.ShapeDtypeStruct((n, cout), jnp.float32),
                   jax.ShapeDtypeStruct((2, cout), jnp.float32)],
    )(xg, nbr, w, b.reshape(1, cout))


_BO = 1024


def _out_body(mx_ref, mn_ref, st_ref, g_ref, be_ref, o_ref):
    st = st_ref[...]
    nk = float(NPTS * KNBR)
    mu = st[0:1] / nk
    var = st[1:2] / nk - mu * mu
    g = g_ref[...]
    m = jnp.where(g >= 0, mx_ref[...], mn_ref[...])
    z = (m - mu) / jnp.sqrt(var + EPS) * g + be_ref[...]
    o_ref[...] = jnp.where(z >= 0, z, SLOPE * z)


def _edge_out(mx, mn, st, g, be):
    n, cout = mx.shape
    row = pl.BlockSpec((_BO, cout), lambda i: (i, 0))
    vec = pl.BlockSpec((1, cout), lambda i: (0, 0))
    return pl.pallas_call(
        _out_body,
        grid=(n // _BO,),
        in_specs=[row, row, pl.BlockSpec((2, cout), lambda i: (0, 0)),
                  vec, vec],
        out_specs=row,
        out_shape=jax.ShapeDtypeStruct((n, cout), jnp.float32),
    )(mx, mn, st, g.reshape(1, cout), be.reshape(1, cout))


def _final_linear_body(x1_ref, x2_ref, x3_ref, x4_ref, w_ref, b_ref, o_ref):
    feat = jnp.concatenate(
        [x1_ref[...], x2_ref[...], x3_ref[...], x4_ref[...]], axis=1)
    o_ref[...] = lax.dot_general(
        feat, w_ref[...], (((1,), (0,)), ((), ())),
        preferred_element_type=jnp.float32) + b_ref[...]


def _final_linear(x1, x2, x3, x4, w5, b5, br=512):
    n = x1.shape[0]
    m = w5.shape[1]
    cat = w5.shape[0]

    def rowspec(c):
        return pl.BlockSpec((br, c), lambda i: (i, 0))

    return pl.pallas_call(
        _final_linear_body,
        grid=(n // br,),
        in_specs=[rowspec(x1.shape[1]), rowspec(x2.shape[1]),
                  rowspec(x3.shape[1]), rowspec(x4.shape[1]),
                  pl.BlockSpec((cat, m), lambda i: (0, 0)),
                  pl.BlockSpec((1, m), lambda i: (0, 0))],
        out_specs=pl.BlockSpec((br, m), lambda i: (i, 0)),
        out_shape=jax.ShapeDtypeStruct((n, m), jnp.float32),
    )(x1, x2, x3, x4, w5, b5.reshape(1, m))


def _edge_layer(x, w, bias, g, be):
    n, cin = x.shape
    if cin < GW:
        xg = jnp.concatenate(
            [x, jnp.zeros((n, GW - cin), jnp.float32)], axis=1)
    else:
        xg = x
    idx = _knn(xg, xg.T)
    nbr = _sc_gather(xg, idx.reshape(-1))
    mx, mn, st = _edge_fused(xg, nbr, w, bias, cin)
    return _edge_out(mx, mn, st, g, be)


def kernel(x, W1, b1, g1, be1, W2, b2, g2, be2, W3, b3, g3, be3,
           W4, b4, g4, be4, W5, b5):
    x1 = _edge_layer(x, W1, b1, g1, be1)
    x2 = _edge_layer(x1, W2, b2, g2, be2)
    x3 = _edge_layer(x2, W3, b3, g3, be3)
    x4 = _edge_layer(x3, W4, b4, g4, be4)
    return _final_linear(x1, x2, x3, x4, W5, b5)


# speedup vs baseline: 2.6007x; 1.1286x over previous
import functools

import jax
import jax.numpy as jnp
from jax import lax
from jax.experimental import pallas as pl
from jax.experimental.pallas import tpu as pltpu
from jax.experimental.pallas import tpu_sc as plsc

NPTS = 8192
KNBR = 20
SLOPE = 0.2
EPS = 1e-5
GW = 128


_BR = 256


_NG = 512
_NT = NPTS // _NG
_RS = 16
_INF = jnp.inf


def _insert3(d, c, r1, i1, r2, i2, r3, i3):
    c1 = d < r1
    c2 = d < r2
    c3 = d < r3
    r1n = jnp.where(c1, d, r1)
    i1n = jnp.where(c1, c, i1)
    r2n = jnp.where(c1, r1, jnp.where(c2, d, r2))
    i2n = jnp.where(c1, i1, jnp.where(c2, c, i2))
    r3n = jnp.where(c2, r2, jnp.where(c3, d, r3))
    i3n = jnp.where(c2, i2, jnp.where(c3, c, i3))
    return r1n, i1n, r2n, i2n, r3n, i3n


def _knn_body(x_ref, xt_ref, idx_ref, d_ref):
    xt = xt_ref[...]
    xb = x_ref[...]
    dot = lax.dot_general(
        xb, xt, (((1,), (0,)), ((), ())),
        preferred_element_type=jnp.float32)
    sqr = jnp.sum(xb * xb, axis=1, keepdims=True)
    sqc = jnp.sum(xt * xt, axis=0, keepdims=True)
    d = (sqr - 2.0 * dot) + sqc
    d_ref[...] = d

    giota = lax.broadcasted_iota(jnp.int32, (_RS, _NG), 1)
    oiota = lax.broadcasted_iota(jnp.int32, (_RS, KNBR), 1)
    shape = (_RS, _NG)

    slices = []
    for rs in range(0, _BR, _RS):
        r1 = jnp.full(shape, _INF, jnp.float32)
        r2 = jnp.full(shape, _INF, jnp.float32)
        r3 = jnp.full(shape, _INF, jnp.float32)
        i1 = jnp.zeros(shape, jnp.int32)
        i2 = jnp.zeros(shape, jnp.int32)
        i3 = jnp.zeros(shape, jnp.int32)
        for t in range(_NT):
            dt = d_ref[rs:rs + _RS, t * _NG:(t + 1) * _NG]
            ct = giota + t * _NG
            r1, i1, r2, i2, r3, i3 = _insert3(dt, ct, r1, i1, r2, i2, r3, i3)

        outs = []
        trig = jnp.int32(0)
        for _ in range(KNBR):
            m = jnp.min(r1, axis=1, keepdims=True)
            istar = jnp.min(jnp.where(r1 == m, i1, NPTS), axis=1,
                            keepdims=True)
            outs.append(istar)
            sel = (r1 == m) & (i1 == istar)
            trig = jnp.maximum(
                trig, jnp.max(jnp.where(sel & (r2 == _INF), 1, 0)))
            r1 = jnp.where(sel, r2, r1)
            i1 = jnp.where(sel, i2, i1)
            r2 = jnp.where(sel, r3, r2)
            i2 = jnp.where(sel, i3, i2)
            r3 = jnp.where(sel, _INF, r3)
        slices.append((rs, jnp.concatenate(outs, axis=1), trig))

    for rs, idx_slice, trig in slices:
        def _fallback(rs=rs):
            cols = lax.broadcasted_iota(jnp.int32, (_RS, NPTS), 1)

            def fb(j, carry):
                dd, out = carry
                mm = jnp.min(dd, axis=1, keepdims=True)
                ii = jnp.min(jnp.where(dd == mm, cols, NPTS), axis=1,
                             keepdims=True)
                out = jnp.where(oiota == j, ii, out)
                dd = jnp.where(cols == ii, _INF, dd)
                return dd, out

            dd0 = d_ref[rs:rs + _RS, :]
            _, out = lax.fori_loop(
                0, KNBR, fb, (dd0, jnp.zeros((_RS, KNBR), jnp.int32)))
            return out

        idx_slice = lax.cond(trig > 0, _fallback,
                             lambda idx_slice=idx_slice: idx_slice)
        idx_ref[rs:rs + _RS, :] = idx_slice


def _knn(xp, xt):
    n, c = xp.shape
    return pl.pallas_call(
        _knn_body,
        grid=(n // _BR,),
        in_specs=[pl.BlockSpec((_BR, c), lambda i: (i, 0)),
                  pl.BlockSpec((c, n), lambda i: (0, 0))],
        out_specs=pl.BlockSpec((_BR, KNBR), lambda i: (i, 0)),
        out_shape=jax.ShapeDtypeStruct((n, KNBR), jnp.int32),
        scratch_shapes=[pltpu.VMEM((_BR, n), jnp.float32)],
    )(xp, xt)


def _sc_gather(table, idx_flat):
    n, width = table.shape
    ne = n * KNBR
    info = plsc.get_sparse_core_info()
    ncores = info.num_cores
    nw = ncores * info.num_subcores
    per_w = ne // nw
    eb = 512
    nch = per_w // eb
    mesh = plsc.VectorSubcoreMesh(core_axis_name="c", subcore_axis_name="s")

    @functools.partial(
        pl.kernel, mesh=mesh,
        out_type=jax.ShapeDtypeStruct((ne, width), jnp.float32),
        scratch_types=[
            pltpu.VMEM((eb,), jnp.int32),
            pltpu.VMEM((eb, width), jnp.float32),
            pltpu.SemaphoreType.DMA,
        ])
    def k(tab_hbm, idx_hbm, out_hbm, idx_v, rows_v, sem):
        wid = lax.axis_index("s") * ncores + lax.axis_index("c")

        def chunk_body(ci, carry):
            base = wid * per_w + ci * eb
            pltpu.sync_copy(idx_hbm.at[pl.ds(base, eb)], idx_v)
            pltpu.async_copy(tab_hbm.at[idx_v], rows_v, sem).wait()
            pltpu.sync_copy(rows_v, out_hbm.at[pl.ds(base, eb)])
            return carry

        lax.fori_loop(0, nch, chunk_body, 0)

    return k(table, idx_flat)


def _edge_body(cin, x_ref, nbr_ref, w_ref, b_ref, mx_ref, mn_ref, st_ref):
    xs = x_ref[...][:, :cin]
    nb = nbr_ref[...][:, :cin]
    crep = jnp.broadcast_to(
        jnp.reshape(xs, (_BR, 1, cin)), (_BR, KNBR, cin))
    crep = jnp.reshape(crep, (_BR * KNBR, cin))
    e_in = jnp.concatenate([crep, nb - crep], axis=1)
    e = lax.dot_general(
        e_in, w_ref[...], (((1,), (0,)), ((), ())),
        preferred_element_type=jnp.float32) + b_ref[...]

    @pl.when(pl.program_id(0) == 0)
    def _():
        st_ref[...] = jnp.zeros_like(st_ref)

    st_ref[...] += jnp.concatenate(
        [jnp.sum(e, axis=0, keepdims=True),
         jnp.sum(e * e, axis=0, keepdims=True)], axis=0)
    e3 = jnp.reshape(e, (_BR, KNBR, e.shape[1]))
    mx_ref[...] = jnp.max(e3, axis=1)
    mn_ref[...] = jnp.min(e3, axis=1)


def _edge_fused(xg, nbr, w, b, cin):
    n = xg.shape[0]
    cout = w.shape[1]
    body = functools.partial(_edge_body, cin)
    return pl.pallas_call(
        body,
        grid=(n // _BR,),
        in_specs=[pl.BlockSpec((_BR, GW), lambda i: (i, 0)),
                  pl.BlockSpec((_BR * KNBR, GW), lambda i: (i, 0)),
                  pl.BlockSpec((2 * cin, cout), lambda i: (0, 0)),
                  pl.BlockSpec((1, cout), lambda i: (0, 0))],
        out_specs=[pl.BlockSpec((_BR, cout), lambda i: (i, 0)),
                   pl.BlockSpec((_BR, cout), lambda i: (i, 0)),
                   pl.BlockSpec((2, cout), lambda i: (0, 0))],
        out_shape=[jax.ShapeDtypeStruct((n, cout), jnp.float32),
                   jax.ShapeDtypeStruct((n, cout), jnp.float32),
                   jax.ShapeDtypeStruct((2, cout), jnp.float32)],
    )(xg, nbr, w, b.reshape(1, cout))


_BO = 1024


def _out_body(mx_ref, mn_ref, st_ref, g_ref, be_ref, o_ref):
    st = st_ref[...]
    nk = float(NPTS * KNBR)
    mu = st[0:1] / nk
    var = st[1:2] / nk - mu * mu
    g = g_ref[...]
    m = jnp.where(g >= 0, mx_ref[...], mn_ref[...])
    z = (m - mu) / jnp.sqrt(var + EPS) * g + be_ref[...]
    o_ref[...] = jnp.where(z >= 0, z, SLOPE * z)


def _edge_out(mx, mn, st, g, be):
    n, cout = mx.shape
    row = pl.BlockSpec((_BO, cout), lambda i: (i, 0))
    vec = pl.BlockSpec((1, cout), lambda i: (0, 0))
    return pl.pallas_call(
        _out_body,
        grid=(n // _BO,),
        in_specs=[row, row, pl.BlockSpec((2, cout), lambda i: (0, 0)),
                  vec, vec],
        out_specs=row,
        out_shape=jax.ShapeDtypeStruct((n, cout), jnp.float32),
    )(mx, mn, st, g.reshape(1, cout), be.reshape(1, cout))


def _final_linear_body(x1_ref, x2_ref, x3_ref, x4_ref, w_ref, b_ref, o_ref):
    feat = jnp.concatenate(
        [x1_ref[...], x2_ref[...], x3_ref[...], x4_ref[...]], axis=1)
    o_ref[...] = lax.dot_general(
        feat, w_ref[...], (((1,), (0,)), ((), ())),
        preferred_element_type=jnp.float32) + b_ref[...]


def _final_linear(x1, x2, x3, x4, w5, b5, br=512):
    n = x1.shape[0]
    m = w5.shape[1]
    cat = w5.shape[0]

    def rowspec(c):
        return pl.BlockSpec((br, c), lambda i: (i, 0))

    return pl.pallas_call(
        _final_linear_body,
        grid=(n // br,),
        in_specs=[rowspec(x1.shape[1]), rowspec(x2.shape[1]),
                  rowspec(x3.shape[1]), rowspec(x4.shape[1]),
                  pl.BlockSpec((cat, m), lambda i: (0, 0)),
                  pl.BlockSpec((1, m), lambda i: (0, 0))],
        out_specs=pl.BlockSpec((br, m), lambda i: (i, 0)),
        out_shape=jax.ShapeDtypeStruct((n, m), jnp.float32),
    )(x1, x2, x3, x4, w5, b5.reshape(1, m))


def _edge_layer(x, w, bias, g, be):
    n, cin = x.shape
    if cin < GW:
        xg = jnp.concatenate(
            [x, jnp.zeros((n, GW - cin), jnp.float32)], axis=1)
    else:
        xg = x
    idx = _knn(xg, xg.T)
    nbr = _sc_gather(xg, idx.reshape(-1))
    mx, mn, st = _edge_fused(xg, nbr, w, bias, cin)
    return _edge_out(mx, mn, st, g, be)


def kernel(x, W1, b1, g1, be1, W2, b2, g2, be2, W3, b3, g3, be3,
           W4, b4, g4, be4, W5, b5):
    x1 = _edge_layer(x, W1, b1, g1, be1)
    x2 = _edge_layer(x1, W2, b2, g2, be2)
    x3 = _edge_layer(x2, W3, b3, g3, be3)
    x4 = _edge_layer(x3, W4, b4, g4, be4)
    return _final_linear(x1, x2, x3, x4, W5, b5)
